# baseline (speedup 1.0000x reference)
import jax
import jax.numpy as jnp
from jax.experimental import pallas as pl
from jax.experimental.pallas import tpu as pltpu

K = 20
EPS = 1e-5
B = 8
N = 1024
NEG = -3.0e38


def _lrelu(v):
    return jnp.where(v >= 0, v, 0.2 * v)


PB = 1


def _gcn_body(x_ref, w1, w2, w3, w4, w5,
              s1, t1, s2, t2, s3, t3, s4, t4, s5, t5, out_ref):
    lane = jax.lax.broadcasted_iota(jnp.int32, (N, N), 1)
    sub = jax.lax.broadcasted_iota(jnp.int32, (N, N), 0)

    xbs, keys = [], []
    for p in range(PB):
        xb = x_ref[p]
        g = jax.lax.dot_general(xb, xb, (((0,), (0,)), ((), ())),
                                preferred_element_type=jnp.float32)
        xx = jnp.sum(xb * xb, axis=0)
        pd = 2.0 * g - xx[:, None] - xx[None, :]
        bits = jax.lax.bitcast_convert_type(pd + 0.0, jnp.int32)
        key = bits ^ (jax.lax.shift_right_arithmetic(bits, 31)
                      & jnp.int32(0x7FFFFFFF))
        xbs.append(xb)
        keys.append(key)

    def count_ge(arr, cand, r):
        c = jnp.sum((arr >= cand).astype(jnp.int32), axis=1, keepdims=True)
        return c >= r

    khis, klos = [], []
    for key in keys:
        khis.append(jax.lax.shift_right_arithmetic(key, 16)
                    .astype(jnp.int16))
        klos.append(((key & 0xFFFF) - 32768).astype(jnp.int16))

    h0s = []
    for khi in khis:
        ge = count_ge(khi, jnp.int16(0), K)
        h0s.append(jnp.where(ge, 0, -32768).astype(jnp.int16))

    def hstep(i, hs):
        out = []
        for khi, h_c in zip(khis, hs):
            cand = (h_c.astype(jnp.int32)
                    | jnp.left_shift(jnp.int32(1), 14 - i)).astype(jnp.int16)
            out.append(jnp.where(count_ge(khi, cand, K), cand, h_c))
        return tuple(out)

    hss = jax.lax.fori_loop(0, 15, hstep, tuple(h0s))

    ts = []
    for p in range(PB):
        khi, klo, hst = khis[p], klos[p], hss[p]
        ngt_hi = jnp.sum((khi > hst).astype(jnp.int32), axis=1, keepdims=True)
        r = K - ngt_hi
        mlo = jnp.where(khi == hst, klo, jnp.int16(-32768))
        ge = count_ge(mlo, jnp.int16(0), r)
        l0 = jnp.where(ge, 0, -32768).astype(jnp.int16)

        def lstep(i, l_c, mlo=mlo, r=r):
            cand = (l_c.astype(jnp.int32)
                    | jnp.left_shift(jnp.int32(1), 14 - i)).astype(jnp.int16)
            return jnp.where(count_ge(mlo, cand, r), cand, l_c)

        lst = jax.lax.fori_loop(0, 15, lstep, l0)
        ts.append((hst.astype(jnp.int32) << 16)
                  | (lst.astype(jnp.int32) + 32768))

    ltri = (sub < lane).astype(jnp.bfloat16)
    for p in range(PB):
        key, t = keys[p], ts[p]
        gt = key > t
        eqm = key == t
        need = K - jnp.sum(gt.astype(jnp.int32), axis=1, keepdims=True)
        ranks = jax.lax.dot_general(eqm.astype(jnp.bfloat16), ltri,
                                    (((1,), (0,)), ((), ())),
                                    preferred_element_type=jnp.float32)
        a = (gt | (eqm & (ranks < need.astype(jnp.float32))))
        a = a.astype(jnp.float32)
        a = jnp.where(lane == sub, 1.0, a)
        dinv = jax.lax.rsqrt(jnp.sum(a, axis=0))
        a_hat = a * dinv[:, None] * dinv[None, :]

        def layer(hw, s, t_b):
            agg = jax.lax.dot_general(a_hat, hw, (((0,), (0,)), ((), ())),
                                      preferred_element_type=jnp.float32)
            return _lrelu(agg * s[...] + t_b[...])

        hw = jax.lax.dot_general(xbs[p], w1[...], (((0,), (0,)), ((), ())),
                                 preferred_element_type=jnp.float32)
        h = layer(hw, s1, t1)
        h = layer(jnp.dot(h, w2[...], preferred_element_type=jnp.float32), s2, t2)
        h = layer(jnp.dot(h, w3[...], preferred_element_type=jnp.float32), s3, t3)
        h = layer(jnp.dot(h, w4[...], preferred_element_type=jnp.float32), s4, t4)
        h = layer(jnp.dot(h, w5[...], preferred_element_type=jnp.float32), s5, t5)
        out_ref[p, 0] = jnp.sum(h, axis=0)


def _head_body(s_ref, l1, s6, t6, l2, s7, t7, l3, t8, out_ref):
    s = s_ref[...]
    y = (jnp.dot(s * (1.0 / N), l1[:N, :],
                 preferred_element_type=jnp.float32)
         + jnp.dot(s, l1[N:, :], preferred_element_type=jnp.float32))
    y = _lrelu(y * s6[...] + t6[...])
    y = _lrelu(jnp.dot(y, l2[...], preferred_element_type=jnp.float32)
               * s7[...] + t7[...])
    out_ref[...] = jnp.dot(y, l3[...], preferred_element_type=jnp.float32) + t8[...]


def kernel(x, W1, b1, W2, b2, W3, b3, W4, b4, W5, b5,
           g1, be1, g2, be2, g3, be3, g4, be4, g5, be5, g6, be6, g7, be7,
           L1W, L2W, L2b, L3W, L3b):
    inv = jnp.float32(1.0 / jnp.sqrt(1.0 + EPS))

    def fuse(gv, bev, bv=None):
        s = (gv * inv).reshape(1, -1)
        t = (bev if bv is None else bv * gv * inv + bev).reshape(1, -1)
        return s, t

    s1, t1 = fuse(g1, be1, b1)
    s2, t2 = fuse(g2, be2, b2)
    s3, t3 = fuse(g3, be3, b3)
    s4, t4 = fuse(g4, be4, b4)
    s5, t5 = fuse(g5, be5, b5)
    s6, t6 = fuse(g6, be6)
    s7, t7 = fuse(g7, be7, L2b)
    t8 = L3b.reshape(1, -1)

    dims = [64, 128, 256, 512, 1024]
    full = lambda a: pl.BlockSpec(a.shape, lambda b: (0,) * a.ndim)
    vec_specs = []
    for d in dims:
        vec_specs += [pl.BlockSpec((1, d), lambda b: (0, 0))] * 2

    pooled = pl.pallas_call(
        _gcn_body,
        grid=(B // PB,),
        in_specs=[pl.BlockSpec((PB, 3, N), lambda b: (b, 0, 0)),
                  full(W1), full(W2), full(W3), full(W4), full(W5)] + vec_specs,
        out_specs=pl.BlockSpec((PB, 1, N), lambda b: (b, 0, 0)),
        out_shape=jax.ShapeDtypeStruct((B, 1, N), jnp.float32),
        compiler_params=pltpu.CompilerParams(
            dimension_semantics=("parallel",)),
    )(x, W1, W2, W3, W4, W5, s1, t1, s2, t2, s3, t3, s4, t4, s5, t5)
    pooled = pooled.reshape(B, N)

    out = pl.pallas_call(
        _head_body,
        out_shape=jax.ShapeDtypeStruct((B, 40), jnp.float32),
    )(pooled, L1W, s6, t6, L2W, s7, t7, L3W, t8)
    return out


# speedup vs baseline: 1.4407x; 1.4407x over previous
import jax
import jax.numpy as jnp
from jax.experimental import pallas as pl
from jax.experimental.pallas import tpu as pltpu

K = 20
EPS = 1e-5
B = 8
N = 1024
NEG = -3.0e38


def _lrelu(v):
    return jnp.where(v >= 0, v, 0.2 * v)


PB = 1


def _gcn_body(x_ref, w1, w2, w3, w4, w5,
              s1, t1, s2, t2, s3, t3, s4, t4, s5, t5, out_ref):
    lane = jax.lax.broadcasted_iota(jnp.int32, (N, N), 1)
    sub = jax.lax.broadcasted_iota(jnp.int32, (N, N), 0)

    xbs, keys = [], []
    for p in range(PB):
        xb = x_ref[p]
        g = jax.lax.dot_general(xb, xb, (((0,), (0,)), ((), ())),
                                preferred_element_type=jnp.float32)
        xx = jnp.sum(xb * xb, axis=0)
        pd = 2.0 * g - xx[:, None] - xx[None, :]
        bits = jax.lax.bitcast_convert_type(pd + 0.0, jnp.int32)
        key = bits ^ (jax.lax.shift_right_arithmetic(bits, 31)
                      & jnp.int32(0x7FFFFFFF))
        xbs.append(xb)
        keys.append(key)

    t0s = []
    for key in keys:
        cnt0 = jnp.sum((key >= 0).astype(jnp.int32), axis=1, keepdims=True)
        t0s.append(jnp.where(cnt0 >= K, 0, jnp.int32(-2147483648)))

    def bstep(i, ts_c):
        shift = jnp.left_shift(jnp.int32(1), 30 - i)
        out = []
        for key, t_c in zip(keys, ts_c):
            cand = t_c | shift
            cnt = jnp.sum((key >= cand).astype(jnp.int32),
                          axis=1, keepdims=True)
            out.append(jnp.where(cnt >= K, cand, t_c))
        return tuple(out)

    ts = jax.lax.fori_loop(0, 31, bstep, tuple(t0s))

    ltri = (sub < lane).astype(jnp.bfloat16)
    for p in range(PB):
        key, t = keys[p], ts[p]
        gt = key > t
        eqm = key == t
        need = K - jnp.sum(gt.astype(jnp.int32), axis=1, keepdims=True)
        ranks = jax.lax.dot_general(eqm.astype(jnp.bfloat16), ltri,
                                    (((1,), (0,)), ((), ())),
                                    preferred_element_type=jnp.float32)
        a = (gt | (eqm & (ranks < need.astype(jnp.float32))))
        a = a.astype(jnp.float32)
        a = jnp.where(lane == sub, 1.0, a)
        dinv = jax.lax.rsqrt(jnp.sum(a, axis=0))
        a_hat = a * dinv[:, None] * dinv[None, :]

        def layer(hw, s, t_b):
            agg = jax.lax.dot_general(a_hat, hw, (((0,), (0,)), ((), ())),
                                      preferred_element_type=jnp.float32)
            return _lrelu(agg * s[...] + t_b[...])

        hw = jax.lax.dot_general(xbs[p], w1[...], (((0,), (0,)), ((), ())),
                                 preferred_element_type=jnp.float32)
        h = layer(hw, s1, t1)
        h = layer(jnp.dot(h, w2[...], preferred_element_type=jnp.float32), s2, t2)
        h = layer(jnp.dot(h, w3[...], preferred_element_type=jnp.float32), s3, t3)
        h = layer(jnp.dot(h, w4[...], preferred_element_type=jnp.float32), s4, t4)
        h = layer(jnp.dot(h, w5[...], preferred_element_type=jnp.float32), s5, t5)
        out_ref[p, 0] = jnp.sum(h, axis=0)


def _head_body(s_ref, l1, s6, t6, l2, s7, t7, l3, t8, out_ref):
    s = s_ref[...]
    y = (jnp.dot(s * (1.0 / N), l1[:N, :],
                 preferred_element_type=jnp.float32)
         + jnp.dot(s, l1[N:, :], preferred_element_type=jnp.float32))
    y = _lrelu(y * s6[...] + t6[...])
    y = _lrelu(jnp.dot(y, l2[...], preferred_element_type=jnp.float32)
               * s7[...] + t7[...])
    out_ref[...] = jnp.dot(y, l3[...], preferred_element_type=jnp.float32) + t8[...]


def kernel(x, W1, b1, W2, b2, W3, b3, W4, b4, W5, b5,
           g1, be1, g2, be2, g3, be3, g4, be4, g5, be5, g6, be6, g7, be7,
           L1W, L2W, L2b, L3W, L3b):
    inv = jnp.float32(1.0 / jnp.sqrt(1.0 + EPS))

    def fuse(gv, bev, bv=None):
        s = (gv * inv).reshape(1, -1)
        t = (bev if bv is None else bv * gv * inv + bev).reshape(1, -1)
        return s, t

    s1, t1 = fuse(g1, be1, b1)
    s2, t2 = fuse(g2, be2, b2)
    s3, t3 = fuse(g3, be3, b3)
    s4, t4 = fuse(g4, be4, b4)
    s5, t5 = fuse(g5, be5, b5)
    s6, t6 = fuse(g6, be6)
    s7, t7 = fuse(g7, be7, L2b)
    t8 = L3b.reshape(1, -1)

    dims = [64, 128, 256, 512, 1024]
    full = lambda a: pl.BlockSpec(a.shape, lambda b: (0,) * a.ndim)
    vec_specs = []
    for d in dims:
        vec_specs += [pl.BlockSpec((1, d), lambda b: (0, 0))] * 2

    pooled = pl.pallas_call(
        _gcn_body,
        grid=(B // PB,),
        in_specs=[pl.BlockSpec((PB, 3, N), lambda b: (b, 0, 0)),
                  full(W1), full(W2), full(W3), full(W4), full(W5)] + vec_specs,
        out_specs=pl.BlockSpec((PB, 1, N), lambda b: (b, 0, 0)),
        out_shape=jax.ShapeDtypeStruct((B, 1, N), jnp.float32),
        compiler_params=pltpu.CompilerParams(
            dimension_semantics=("parallel",)),
    )(x, W1, W2, W3, W4, W5, s1, t1, s2, t2, s3, t3, s4, t4, s5, t5)
    pooled = pooled.reshape(B, N)

    out = pl.pallas_call(
        _head_body,
        out_shape=jax.ShapeDtypeStruct((B, 40), jnp.float32),
    )(pooled, L1W, s6, t6, L2W, s7, t7, L3W, t8)
    return out


# speedup vs baseline: 1.4929x; 1.0362x over previous
import jax
import jax.numpy as jnp
from jax.experimental import pallas as pl
from jax.experimental.pallas import tpu as pltpu

K = 20
EPS = 1e-5
B = 8
N = 1024
NEG = -3.0e38


def _lrelu(v):
    return jnp.where(v >= 0, v, 0.2 * v)


PB = 1


def _gcn_body(x_ref, w1, w2, w3, w4, w5,
              s1, t1, s2, t2, s3, t3, s4, t4, s5, t5, out_ref, a_scr):
    lane = jax.lax.broadcasted_iota(jnp.int32, (N, N), 1)
    sub = jax.lax.broadcasted_iota(jnp.int32, (N, N), 0)

    xbs, keys = [], []
    for p in range(PB):
        xb = x_ref[p]
        g = jax.lax.dot_general(xb, xb, (((0,), (0,)), ((), ())),
                                preferred_element_type=jnp.float32)
        xx = jnp.sum(xb * xb, axis=0)
        pd = 2.0 * g - xx[:, None] - xx[None, :]
        bits = jax.lax.bitcast_convert_type(pd + 0.0, jnp.int32)
        key = bits ^ (jax.lax.shift_right_arithmetic(bits, 31)
                      & jnp.int32(0x7FFFFFFF))
        xbs.append(xb)
        keys.append(key)

    t0s = []
    for key in keys:
        cnt0 = jnp.sum((key >= 0).astype(jnp.int32), axis=1, keepdims=True)
        t0s.append(jnp.where(cnt0 >= K, 0, jnp.int32(-2147483648)))

    def bstep(i, ts_c):
        shift = jnp.left_shift(jnp.int32(1), 30 - i)
        out = []
        for key, t_c in zip(keys, ts_c):
            cand = t_c | shift
            cnt = jnp.sum((key >= cand).astype(jnp.int32),
                          axis=1, keepdims=True)
            out.append(jnp.where(cnt >= K, cand, t_c))
        return tuple(out)

    ts = jax.lax.fori_loop(0, 31, bstep, tuple(t0s))

    ltri = (sub < lane).astype(jnp.bfloat16)
    for p in range(PB):
        key, t = keys[p], ts[p]
        gt = key > t
        eqm = key == t
        pk = jnp.sum(gt.astype(jnp.int32) + (eqm.astype(jnp.int32) << 11),
                     axis=1, keepdims=True)
        need = K - (pk & 2047)
        ntie = pk >> 11

        a_scr[...] = (gt | eqm).astype(jnp.float32)

        @pl.when(jnp.any(ntie > need))
        def _rank_fix():
            ranks = jax.lax.dot_general(eqm.astype(jnp.bfloat16), ltri,
                                        (((1,), (0,)), ((), ())),
                                        preferred_element_type=jnp.float32)
            sel = gt | (eqm & (ranks < need.astype(jnp.float32)))
            a_scr[...] = sel.astype(jnp.float32)

        a = a_scr[...]
        a = jnp.where(lane == sub, 1.0, a)
        dinv = jax.lax.rsqrt(jnp.sum(a, axis=0))
        a_hat = a * dinv[:, None] * dinv[None, :]

        def layer(hw, s, t_b):
            agg = jax.lax.dot_general(a_hat, hw, (((0,), (0,)), ((), ())),
                                      preferred_element_type=jnp.float32)
            return _lrelu(agg * s[...] + t_b[...])

        hw = jax.lax.dot_general(xbs[p], w1[...], (((0,), (0,)), ((), ())),
                                 preferred_element_type=jnp.float32)
        h = layer(hw, s1, t1)
        h = layer(jnp.dot(h, w2[...], preferred_element_type=jnp.float32), s2, t2)
        h = layer(jnp.dot(h, w3[...], preferred_element_type=jnp.float32), s3, t3)
        h = layer(jnp.dot(h, w4[...], preferred_element_type=jnp.float32), s4, t4)
        h = layer(jnp.dot(h, w5[...], preferred_element_type=jnp.float32), s5, t5)
        out_ref[p, 0] = jnp.sum(h, axis=0)


def _head_body(s_ref, l1, s6, t6, l2, s7, t7, l3, t8, out_ref):
    s = s_ref[...]
    y = (jnp.dot(s * (1.0 / N), l1[:N, :],
                 preferred_element_type=jnp.float32)
         + jnp.dot(s, l1[N:, :], preferred_element_type=jnp.float32))
    y = _lrelu(y * s6[...] + t6[...])
    y = _lrelu(jnp.dot(y, l2[...], preferred_element_type=jnp.float32)
               * s7[...] + t7[...])
    out_ref[...] = jnp.dot(y, l3[...], preferred_element_type=jnp.float32) + t8[...]


def kernel(x, W1, b1, W2, b2, W3, b3, W4, b4, W5, b5,
           g1, be1, g2, be2, g3, be3, g4, be4, g5, be5, g6, be6, g7, be7,
           L1W, L2W, L2b, L3W, L3b):
    inv = jnp.float32(1.0 / jnp.sqrt(1.0 + EPS))

    def fuse(gv, bev, bv=None):
        s = (gv * inv).reshape(1, -1)
        t = (bev if bv is None else bv * gv * inv + bev).reshape(1, -1)
        return s, t

    s1, t1 = fuse(g1, be1, b1)
    s2, t2 = fuse(g2, be2, b2)
    s3, t3 = fuse(g3, be3, b3)
    s4, t4 = fuse(g4, be4, b4)
    s5, t5 = fuse(g5, be5, b5)
    s6, t6 = fuse(g6, be6)
    s7, t7 = fuse(g7, be7, L2b)
    t8 = L3b.reshape(1, -1)

    dims = [64, 128, 256, 512, 1024]
    full = lambda a: pl.BlockSpec(a.shape, lambda b: (0,) * a.ndim)
    vec_specs = []
    for d in dims:
        vec_specs += [pl.BlockSpec((1, d), lambda b: (0, 0))] * 2

    pooled = pl.pallas_call(
        _gcn_body,
        grid=(B // PB,),
        in_specs=[pl.BlockSpec((PB, 3, N), lambda b: (b, 0, 0)),
                  full(W1), full(W2), full(W3), full(W4), full(W5)] + vec_specs,
        out_specs=pl.BlockSpec((PB, 1, N), lambda b: (b, 0, 0)),
        out_shape=jax.ShapeDtypeStruct((B, 1, N), jnp.float32),
        scratch_shapes=[pltpu.VMEM((N, N), jnp.float32)],
        compiler_params=pltpu.CompilerParams(
            dimension_semantics=("parallel",)),
    )(x, W1, W2, W3, W4, W5, s1, t1, s2, t2, s3, t3, s4, t4, s5, t5)
    pooled = pooled.reshape(B, N)

    out = pl.pallas_call(
        _head_body,
        out_shape=jax.ShapeDtypeStruct((B, 40), jnp.float32),
    )(pooled, L1W, s6, t6, L2W, s7, t7, L3W, t8)
    return out


# speedup vs baseline: 1.4951x; 1.0015x over previous
import jax
import jax.numpy as jnp
from jax.experimental import pallas as pl
from jax.experimental.pallas import tpu as pltpu

K = 20
EPS = 1e-5
B = 8
N = 1024


def _lrelu(v):
    return jnp.where(v >= 0, v, 0.2 * v)


PB = 1


def _gcn_body(x_ref, w1, w2, w3, w4, w5,
              s1, t1, s2, t2, s3, t3, s4, t4, s5, t5, out_ref, a_scr):
    lane = jax.lax.broadcasted_iota(jnp.int32, (N, N), 1)
    sub = jax.lax.broadcasted_iota(jnp.int32, (N, N), 0)

    xbs, keys = [], []
    for p in range(PB):
        xb = x_ref[p]
        g = jax.lax.dot_general(xb, xb, (((0,), (0,)), ((), ())),
                                preferred_element_type=jnp.float32)
        xx = jnp.sum(xb * xb, axis=0)
        pd = 2.0 * g - xx[:, None] - xx[None, :]
        bits = jax.lax.bitcast_convert_type(pd + 0.0, jnp.int32)
        key = bits ^ (jax.lax.shift_right_arithmetic(bits, 31)
                      & jnp.int32(0x7FFFFFFF))
        xbs.append(xb)
        keys.append(key)

    t0s = []
    for key in keys:
        cnt0 = jnp.sum((key >= 0).astype(jnp.int32), axis=1, keepdims=True)
        t0s.append(jnp.where(cnt0 >= K, 0, jnp.int32(-2147483648)))

    def bstep(i, ts_c):
        shift = jnp.left_shift(jnp.int32(1), 30 - i)
        out = []
        for key, t_c in zip(keys, ts_c):
            cand = t_c | shift
            cnt = jnp.sum((key >= cand).astype(jnp.int32),
                          axis=1, keepdims=True)
            out.append(jnp.where(cnt >= K, cand, t_c))
        return tuple(out)

    ts = jax.lax.fori_loop(0, 31, bstep, tuple(t0s))

    ltri = (sub < lane).astype(jnp.bfloat16)
    for p in range(PB):
        key, t = keys[p], ts[p]
        gt = key > t
        eqm = key == t
        pk = jnp.sum(gt.astype(jnp.int32) + (eqm.astype(jnp.int32) << 11),
                     axis=1, keepdims=True)
        need = K - (pk & 2047)
        ntie = pk >> 11

        a_scr[...] = (gt | eqm).astype(jnp.float32)

        @pl.when(jnp.any(ntie > need))
        def _rank_fix():
            ranks = jax.lax.dot_general(eqm.astype(jnp.bfloat16), ltri,
                                        (((1,), (0,)), ((), ())),
                                        preferred_element_type=jnp.float32)
            sel = gt | (eqm & (ranks < need.astype(jnp.float32)))
            a_scr[...] = sel.astype(jnp.float32)

        a = a_scr[...]
        a = jnp.where(lane == sub, 1.0, a)
        dinv = jax.lax.rsqrt(jnp.sum(a, axis=0))
        a_hat = a * dinv[:, None] * dinv[None, :]

        def layer(hw, s, t_b):
            agg = jax.lax.dot_general(a_hat, hw, (((0,), (0,)), ((), ())),
                                      preferred_element_type=jnp.float32)
            return _lrelu(agg * s[...] + t_b[...])

        hw = jax.lax.dot_general(xbs[p], w1[...], (((0,), (0,)), ((), ())),
                                 preferred_element_type=jnp.float32)
        h = layer(hw, s1, t1)
        h = layer(jnp.dot(h, w2[...], preferred_element_type=jnp.float32), s2, t2)
        h = layer(jnp.dot(h, w3[...], preferred_element_type=jnp.float32), s3, t3)
        h = layer(jnp.dot(h, w4[...], preferred_element_type=jnp.float32), s4, t4)
        h = layer(jnp.dot(h, w5[...], preferred_element_type=jnp.float32), s5, t5)
        out_ref[p, 0] = jnp.sum(h, axis=0)


def _head_body(s_ref, l1, s6, t6, l2, s7, t7, l3, t8, out_ref):
    s = s_ref[...]
    y = (jnp.dot(s * (1.0 / N), l1[:N, :],
                 preferred_element_type=jnp.float32)
         + jnp.dot(s, l1[N:, :], preferred_element_type=jnp.float32))
    y = _lrelu(y * s6[...] + t6[...])
    y = _lrelu(jnp.dot(y, l2[...], preferred_element_type=jnp.float32)
               * s7[...] + t7[...])
    out_ref[...] = jnp.dot(y, l3[...], preferred_element_type=jnp.float32) + t8[...]


def kernel(x, W1, b1, W2, b2, W3, b3, W4, b4, W5, b5,
           g1, be1, g2, be2, g3, be3, g4, be4, g5, be5, g6, be6, g7, be7,
           L1W, L2W, L2b, L3W, L3b):
    inv = jnp.float32(1.0 / jnp.sqrt(1.0 + EPS))

    def fuse(gv, bev, bv=None):
        s = (gv * inv).reshape(1, -1)
        t = (bev if bv is None else bv * gv * inv + bev).reshape(1, -1)
        return s, t

    s1, t1 = fuse(g1, be1, b1)
    s2, t2 = fuse(g2, be2, b2)
    s3, t3 = fuse(g3, be3, b3)
    s4, t4 = fuse(g4, be4, b4)
    s5, t5 = fuse(g5, be5, b5)
    s6, t6 = fuse(g6, be6)
    s7, t7 = fuse(g7, be7, L2b)
    t8 = L3b.reshape(1, -1)

    dims = [64, 128, 256, 512, 1024]
    full = lambda a: pl.BlockSpec(a.shape, lambda b: (0,) * a.ndim)
    vec_specs = []
    for d in dims:
        vec_specs += [pl.BlockSpec((1, d), lambda b: (0, 0))] * 2

    pooled = pl.pallas_call(
        _gcn_body,
        grid=(B // PB,),
        in_specs=[pl.BlockSpec((PB, 3, N), lambda b: (b, 0, 0)),
                  full(W1), full(W2), full(W3), full(W4), full(W5)] + vec_specs,
        out_specs=pl.BlockSpec((PB, 1, N), lambda b: (b, 0, 0)),
        out_shape=jax.ShapeDtypeStruct((B, 1, N), jnp.float32),
        scratch_shapes=[pltpu.VMEM((N, N), jnp.float32)],
        compiler_params=pltpu.CompilerParams(
            dimension_semantics=("parallel",)),
    )(x, W1, W2, W3, W4, W5, s1, t1, s2, t2, s3, t3, s4, t4, s5, t5)
    pooled = pooled.reshape(B, N)

    out = pl.pallas_call(
        _head_body,
        out_shape=jax.ShapeDtypeStruct((B, 40), jnp.float32),
    )(pooled, L1W, s6, t6, L2W, s7, t7, L3W, t8)
    return out
